# baseline (speedup 1.0000x reference)
import functools

import jax
import jax.numpy as jnp
from jax import lax
from jax.experimental import pallas as pl
from jax.experimental.pallas import tpu as pltpu
from jax.experimental.pallas import tpu_sc as plsc

N = 10000
E = 320000
D = 128
L3 = 3

NC = 2
NS = 16
NW = NC * NS
K = 80
NCHUNK = 128
EPW = K * NCHUNK
E_PAD = EPW * NW
PAD = E_PAD - E
RPT = 624
ZB = 208
ROWB = K * D * 4
KD = 64
NCHUNK_D = EPW // KD

_sc_mesh = plsc.VectorSubcoreMesh(core_axis_name="c", subcore_axis_name="s")


@functools.partial(
    pl.kernel,
    mesh=_sc_mesh,
    compiler_params=pltpu.CompilerParams(needs_layout_passes=False),
    out_type=jax.ShapeDtypeStruct((E_PAD,), jnp.float32),
    scratch_types=[
        pltpu.VMEM((KD,), jnp.int32),
        pltpu.VMEM((KD,), jnp.int32),
        pltpu.VMEM((KD,), jnp.int32),
        pltpu.VMEM((KD,), jnp.int32),
        pltpu.VMEM((KD, D), jnp.float32),
        pltpu.VMEM((KD, D), jnp.float32),
        pltpu.VMEM((KD, D), jnp.float32),
        pltpu.VMEM((KD, D), jnp.float32),
        pltpu.VMEM((EPW,), jnp.float32),
        pltpu.VMEM((16, 17), jnp.float32),
        pltpu.VMEM_SHARED((N, D), jnp.float32),
        pltpu.SemaphoreType.DMA,
        pltpu.SemaphoreType.DMA,
        pltpu.SemaphoreType.DMA,
        pltpu.SemaphoreType.DMA,
    ],
)
def _dist2_sc(emb, src, dst, d2_out,
              is0, is1, id0, id1, rs0, rs1, rd0, rd1, d2a, tb, emb_sh,
              isem0, isem1, gsem0, gsem1):
    sid = lax.axis_index("s")
    wid = sid * NC + lax.axis_index("c")

    r0 = sid * RPT
    pltpu.sync_copy(emb.at[pl.ds(r0, RPT)], emb_sh.at[pl.ds(r0, RPT)])

    @pl.when(sid == NS - 1)
    def _():
        pltpu.sync_copy(emb.at[pl.ds(NS * RPT, N - NS * RPT)],
                        emb_sh.at[pl.ds(NS * RPT, N - NS * RPT)])

    plsc.subcore_barrier()
    is_ = (is0, is1)
    id_ = (id0, id1)
    rs = (rs0, rs1)
    rd = (rd0, rd1)
    isem = (isem0, isem1)
    gsem = (gsem0, gsem1)
    lanes = lax.iota(jnp.int32, 16)

    def issue_idx(j, t):
        base = wid * EPW + j * KD
        pltpu.async_copy(src.at[pl.ds(base, KD)], is_[t], isem[t])
        pltpu.async_copy(dst.at[pl.ds(base, KD)], id_[t], isem[t])

    def wait_idx(t):
        pltpu.make_async_copy(src.at[pl.ds(0, KD)], is_[t], isem[t]).wait()
        pltpu.make_async_copy(dst.at[pl.ds(0, KD)], id_[t], isem[t]).wait()

    def issue_gather(t):
        pltpu.async_copy(emb_sh.at[is_[t]], rs[t], gsem[t])
        pltpu.async_copy(emb_sh.at[id_[t]], rd[t], gsem[t])

    def wait_gather(t):
        pltpu.make_async_copy(emb_sh.at[is_[t]], rs[t], gsem[t]).wait()
        pltpu.make_async_copy(emb_sh.at[id_[t]], rd[t], gsem[t]).wait()

    def compute(j, t):
        def group_body(g, c2):
            for i in range(16):
                e = g * 16 + i
                acc = jnp.zeros((16,), jnp.float32)
                for c in range(D // 16):
                    sl = pl.ds(c * 16, 16)
                    df = rs[t][e, sl] - rd[t][e, sl]
                    acc = acc + df * df
                plsc.store_scatter(
                    tb, [lanes, jnp.full((16,), i, jnp.int32)], acc)
            s = tb[0, pl.ds(0, 16)]
            for r in range(1, 16):
                s = s + tb[r, pl.ds(0, 16)]
            d2a[pl.ds(j * KD + g * 16, 16)] = s
            return c2

        lax.fori_loop(0, KD // 16, group_body, 0)

    issue_idx(0, 0)
    issue_idx(1, 1)
    wait_idx(0)
    issue_gather(0)
    wait_idx(1)
    issue_gather(1)

    def pair_body(i, c2):
        j0 = 2 * i
        wait_gather(0)
        issue_idx(j0 + 2, 0)
        compute(j0, 0)
        wait_gather(1)
        issue_idx(j0 + 3, 1)
        wait_idx(0)
        issue_gather(0)
        compute(j0 + 1, 1)
        wait_idx(1)
        issue_gather(1)
        return c2

    lax.fori_loop(0, NCHUNK_D // 2 - 1, pair_body, 0)
    wait_gather(0)
    compute(NCHUNK_D - 2, 0)
    wait_gather(1)
    compute(NCHUNK_D - 1, 1)
    pltpu.sync_copy(d2a, d2_out.at[pl.ds(wid * EPW, EPW)])


@functools.partial(
    pl.kernel,
    mesh=_sc_mesh,
    compiler_params=pltpu.CompilerParams(needs_layout_passes=False),
    out_type=jax.ShapeDtypeStruct((NC * N, D), jnp.float32),
    scratch_types=[
        pltpu.VMEM((K,), jnp.int32),
        pltpu.VMEM((K,), jnp.int32),
        pltpu.VMEM((K,), jnp.int32),
        pltpu.VMEM((K,), jnp.int32),
        pltpu.VMEM((K,), jnp.int32),
        pltpu.VMEM((K,), jnp.int32),
        pltpu.VMEM((K,), jnp.float32),
        pltpu.VMEM((K,), jnp.float32),
        pltpu.VMEM((K,), jnp.float32),
        pltpu.VMEM((K,), jnp.float32),
        pltpu.VMEM((K, D), jnp.float32),
        pltpu.VMEM((K, D), jnp.float32),
        pltpu.VMEM((K, D), jnp.float32),
        pltpu.VMEM((K, D), jnp.float32),
        pltpu.VMEM_SHARED((N, D), jnp.float32),
        pltpu.SemaphoreType.DMA,
        pltpu.SemaphoreType.DMA,
        pltpu.SemaphoreType.DMA,
        pltpu.SemaphoreType.DMA,
        pltpu.SemaphoreType.DMA,
        pltpu.SemaphoreType.DMA,
    ],
)
def _agg_sc(hw, src, dst, we, out,
            is0, is1, id0, id1, sx0, sx1, wv0, wv1, ws0, ws1,
            rin0, rin1, rout0, rout1, acc,
            isem0, isem1, gsem0, gsem1, ssem0, ssem1):
    cid = lax.axis_index("c")
    sid = lax.axis_index("s")
    wid = sid * NC + cid
    is_ = (is0, is1)
    id_ = (id0, id1)
    sx = (sx0, sx1)
    wv = (wv0, wv1)
    ws = (ws0, ws1)
    rin = (rin0, rin1)
    rout = (rout0, rout1)
    isem = (isem0, isem1)
    gsem = (gsem0, gsem1)
    ssem = (ssem0, ssem1)

    def zfill(i, c2):
        for c in range(D // 16):
            rout0[i, pl.ds(c * 16, 16)] = jnp.zeros((16,), jnp.float32)
            rout1[i, pl.ds(c * 16, 16)] = jnp.zeros((16,), jnp.float32)
        return c2

    lax.fori_loop(0, K, zfill, 0)
    r0 = sid * RPT
    for t in range(RPT // K):
        pltpu.sync_copy(rout0, acc.at[pl.ds(r0 + t * K, K)])
    pltpu.sync_copy(rout0.at[pl.ds(0, RPT - (RPT // K) * K)],
                    acc.at[pl.ds(r0 + (RPT // K) * K,
                                 RPT - (RPT // K) * K)])

    @pl.when(sid == NS - 1)
    def _():
        pltpu.sync_copy(rout0.at[pl.ds(0, 16)],
                        acc.at[pl.ds(NS * RPT, N - NS * RPT)])

    plsc.subcore_barrier()

    lanes = lax.iota(jnp.int32, 16)
    for t in range(2):
        for q in range(K // 16):
            sx[t][pl.ds(q * 16, 16)] = lanes + q * 16
        pltpu.async_copy(rout[t], acc.at[sx[t]], ssem[t], add=True)

    def issue_idx(j, t):
        base = wid * EPW + j * K
        pltpu.async_copy(src.at[pl.ds(base, K)], is_[t], isem[t])
        pltpu.async_copy(dst.at[pl.ds(base, K)], id_[t], isem[t])
        pltpu.async_copy(we.at[pl.ds(base, K)], wv[t], isem[t])

    def wait_idx(t):
        pltpu.make_async_copy(src.at[pl.ds(0, K)], is_[t], isem[t]).wait()
        pltpu.make_async_copy(dst.at[pl.ds(0, K)], id_[t], isem[t]).wait()
        pltpu.make_async_copy(we.at[pl.ds(0, K)], wv[t], isem[t]).wait()

    def issue_gather(t):
        pltpu.async_copy(hw.at[is_[t]], rin[t], gsem[t])

    def wait_gather(t):
        pltpu.make_async_copy(hw.at[is_[t]], rin[t], gsem[t]).wait()

    def wait_scatter(t):
        pltpu.make_async_copy(rout[t], acc.at[sx[t]], ssem[t]).wait()

    def step(t, nxt):
        wait_gather(t)
        wait_scatter(t)
        for q in range(K // 16):
            sx[t][pl.ds(q * 16, 16)] = id_[t][pl.ds(q * 16, 16)]
            ws[t][pl.ds(q * 16, 16)] = wv[t][pl.ds(q * 16, 16)]
        if nxt is not None:
            issue_idx(nxt, t)

        def group_body(g, c2):
            wvec = ws[t][pl.ds(g * 16, 16)]
            for i in range(16):
                e = g * 16 + i
                w = wvec[i]
                for c in range(D // 16):
                    sl = pl.ds(c * 16, 16)
                    rout[t][e, sl] = rin[t][e, sl] * w
            return c2

        lax.fori_loop(0, K // 16, group_body, 0)
        pltpu.async_copy(rout[t], acc.at[sx[t]], ssem[t], add=True)

    issue_idx(0, 0)
    issue_idx(1, 1)
    wait_idx(0)
    issue_gather(0)

    def pair_body(i, c2):
        j0 = 2 * i
        wait_idx(1)
        issue_gather(1)
        step(0, j0 + 2)
        wait_idx(0)
        issue_gather(0)
        step(1, j0 + 3)
        return c2

    lax.fori_loop(0, NCHUNK // 2 - 1, pair_body, 0)
    wait_idx(1)
    issue_gather(1)
    step(0, None)
    step(1, None)
    wait_scatter(0)
    wait_scatter(1)
    plsc.subcore_barrier()

    pltpu.sync_copy(acc.at[pl.ds(sid * RPT, RPT)],
                    out.at[pl.ds(cid * N + sid * RPT, RPT)])

    @pl.when(sid == NS - 1)
    def _():
        pltpu.sync_copy(acc.at[pl.ds(NS * RPT, N - NS * RPT)],
                        out.at[pl.ds(cid * N + NS * RPT, N - NS * RPT)])


_BN = 2000


def _mm0_body(x_ref, w_ref, o_ref):
    o_ref[...] = jnp.dot(x_ref[...], w_ref[...],
                         preferred_element_type=jnp.float32)


_mm0 = pl.pallas_call(
    _mm0_body,
    grid=(N // _BN,),
    in_specs=[
        pl.BlockSpec((_BN, D), lambda i: (i, 0)),
        pl.BlockSpec((D, D), lambda i: (0, 0)),
    ],
    out_specs=pl.BlockSpec((_BN, D), lambda i: (i, 0)),
    out_shape=jax.ShapeDtypeStruct((N, D), jnp.float32),
)


def _fused_body(p_ref, b_ref, w_ref, o_ref):
    h = jnp.maximum(p_ref[0] + p_ref[1] + b_ref[...], 0.0)
    o_ref[...] = jnp.dot(h, w_ref[...], preferred_element_type=jnp.float32)


_fused = pl.pallas_call(
    _fused_body,
    grid=(N // _BN,),
    in_specs=[
        pl.BlockSpec((2, _BN, D), lambda i: (0, i, 0)),
        pl.BlockSpec((1, D), lambda i: (0, 0)),
        pl.BlockSpec((D, D), lambda i: (0, 0)),
    ],
    out_specs=pl.BlockSpec((_BN, D), lambda i: (i, 0)),
    out_shape=jax.ShapeDtypeStruct((N, D), jnp.float32),
)


def _final_body(p_ref, b_ref, o_ref):
    o_ref[...] = p_ref[0] + p_ref[1] + b_ref[...]


_final = pl.pallas_call(
    _final_body,
    grid=(N // _BN,),
    in_specs=[
        pl.BlockSpec((2, _BN, D), lambda i: (0, i, 0)),
        pl.BlockSpec((1, D), lambda i: (0, 0)),
    ],
    out_specs=pl.BlockSpec((_BN, D), lambda i: (i, 0)),
    out_shape=jax.ShapeDtypeStruct((N, D), jnp.float32),
)


def _wcalc_body(d2_ref, ea_ref, eb_ref, o_ref):
    li = pl.program_id(0)
    dist = jnp.sqrt(d2_ref[...])
    w = jax.nn.sigmoid(ea_ref[li] * dist + eb_ref[li])
    rows_i = lax.broadcasted_iota(jnp.int32, (E_PAD // D, D), 0)
    o_ref[0] = jnp.where(rows_i < E // D, w, 0.0)


_wcalc = pl.pallas_call(
    _wcalc_body,
    grid=(L3,),
    in_specs=[
        pl.BlockSpec((E_PAD // D, D), lambda l: (0, 0)),
        pl.BlockSpec(memory_space=pltpu.SMEM),
        pl.BlockSpec(memory_space=pltpu.SMEM),
    ],
    out_specs=pl.BlockSpec((1, E_PAD // D, D), lambda l: (l, 0, 0)),
    out_shape=jax.ShapeDtypeStruct((L3, E_PAD // D, D), jnp.float32),
)


def kernel(x, adj_t, embeddings, Ws, bs, ea, eb):
    pad = (lax.iota(jnp.int32, PAD) * 97) % N
    src = jnp.concatenate([adj_t[0], pad])
    dst = jnp.concatenate([adj_t[1], pad])

    d2 = _dist2_sc(embeddings, src, dst)
    w3 = _wcalc(d2.reshape(E_PAD // D, D), ea, eb)
    w3 = w3.reshape(L3, E_PAD)

    hw = _mm0(x, Ws[0])
    p = _agg_sc(hw, src, dst, w3[0]).reshape(NC, N, D)
    hw = _fused(p, bs[0].reshape(1, D), Ws[1])
    p = _agg_sc(hw, src, dst, w3[1]).reshape(NC, N, D)
    hw = _fused(p, bs[1].reshape(1, D), Ws[2])
    p = _agg_sc(hw, src, dst, w3[2]).reshape(NC, N, D)
    return _final(p, bs[2].reshape(1, D))


# speedup vs baseline: 1.0337x; 1.0337x over previous
import functools

import jax
import jax.numpy as jnp
from jax import lax
from jax.experimental import pallas as pl
from jax.experimental.pallas import tpu as pltpu
from jax.experimental.pallas import tpu_sc as plsc

N = 10000
E = 320000
D = 128
L3 = 3

NC = 2
NS = 16
NW = NC * NS
K = 80
NCHUNK = 128
EPW = K * NCHUNK
E_PAD = EPW * NW
PAD = E_PAD - E
RPT = 624
ZB = 208
ROWB = K * D * 4
KD = 64
NCHUNK_D = EPW // KD

_sc_mesh = plsc.VectorSubcoreMesh(core_axis_name="c", subcore_axis_name="s")


@functools.partial(
    pl.kernel,
    mesh=_sc_mesh,
    compiler_params=pltpu.CompilerParams(needs_layout_passes=False),
    out_type=jax.ShapeDtypeStruct((E_PAD,), jnp.float32),
    scratch_types=[
        pltpu.VMEM((KD,), jnp.int32),
        pltpu.VMEM((KD,), jnp.int32),
        pltpu.VMEM((KD,), jnp.int32),
        pltpu.VMEM((KD,), jnp.int32),
        pltpu.VMEM((KD, D), jnp.float32),
        pltpu.VMEM((KD, D), jnp.float32),
        pltpu.VMEM((KD, D), jnp.float32),
        pltpu.VMEM((KD, D), jnp.float32),
        pltpu.VMEM((EPW,), jnp.float32),
        pltpu.VMEM((16, 17), jnp.float32),
        pltpu.VMEM_SHARED((N, D), jnp.float32),
        pltpu.SemaphoreType.DMA,
        pltpu.SemaphoreType.DMA,
        pltpu.SemaphoreType.DMA,
        pltpu.SemaphoreType.DMA,
    ],
)
def _dist2_sc(emb, src, dst, d2_out,
              is0, is1, id0, id1, rs0, rs1, rd0, rd1, d2a, tb, emb_sh,
              isem0, isem1, gsem0, gsem1):
    sid = lax.axis_index("s")
    wid = sid * NC + lax.axis_index("c")

    r0 = sid * RPT
    pltpu.sync_copy(emb.at[pl.ds(r0, RPT)], emb_sh.at[pl.ds(r0, RPT)])

    @pl.when(sid == NS - 1)
    def _():
        pltpu.sync_copy(emb.at[pl.ds(NS * RPT, N - NS * RPT)],
                        emb_sh.at[pl.ds(NS * RPT, N - NS * RPT)])

    plsc.subcore_barrier()
    is_ = (is0, is1)
    id_ = (id0, id1)
    rs = (rs0, rs1)
    rd = (rd0, rd1)
    isem = (isem0, isem1)
    gsem = (gsem0, gsem1)
    lanes = lax.iota(jnp.int32, 16)

    def issue_idx(j, t):
        base = wid * EPW + j * KD
        pltpu.async_copy(src.at[pl.ds(base, KD)], is_[t], isem[t])
        pltpu.async_copy(dst.at[pl.ds(base, KD)], id_[t], isem[t])

    def wait_idx(t):
        pltpu.make_async_copy(src.at[pl.ds(0, KD)], is_[t], isem[t]).wait()
        pltpu.make_async_copy(dst.at[pl.ds(0, KD)], id_[t], isem[t]).wait()

    def issue_gather(t):
        pltpu.async_copy(emb_sh.at[is_[t]], rs[t], gsem[t])
        pltpu.async_copy(emb_sh.at[id_[t]], rd[t], gsem[t])

    def wait_gather(t):
        pltpu.make_async_copy(emb_sh.at[is_[t]], rs[t], gsem[t]).wait()
        pltpu.make_async_copy(emb_sh.at[id_[t]], rd[t], gsem[t]).wait()

    def compute(j, t):
        def group_body(g, c2):
            for b in range(4):
                accs = [jnp.zeros((16,), jnp.float32) for _ in range(4)]
                for c in range(D // 16):
                    sl = pl.ds(c * 16, 16)
                    for i in range(4):
                        e = g * 16 + b * 4 + i
                        df = rs[t][e, sl] - rd[t][e, sl]
                        accs[i] = accs[i] + df * df
                for i in range(4):
                    plsc.store_scatter(
                        tb,
                        [lanes, jnp.full((16,), b * 4 + i, jnp.int32)],
                        accs[i])
            s = tb[0, pl.ds(0, 16)]
            for r in range(1, 16):
                s = s + tb[r, pl.ds(0, 16)]
            d2a[pl.ds(j * KD + g * 16, 16)] = s
            return c2

        lax.fori_loop(0, KD // 16, group_body, 0)

    issue_idx(0, 0)
    issue_idx(1, 1)
    wait_idx(0)
    issue_gather(0)
    wait_idx(1)
    issue_gather(1)

    def pair_body(i, c2):
        j0 = 2 * i
        wait_gather(0)
        issue_idx(j0 + 2, 0)
        compute(j0, 0)
        wait_gather(1)
        issue_idx(j0 + 3, 1)
        wait_idx(0)
        issue_gather(0)
        compute(j0 + 1, 1)
        wait_idx(1)
        issue_gather(1)
        return c2

    lax.fori_loop(0, NCHUNK_D // 2 - 1, pair_body, 0)
    wait_gather(0)
    compute(NCHUNK_D - 2, 0)
    wait_gather(1)
    compute(NCHUNK_D - 1, 1)
    pltpu.sync_copy(d2a, d2_out.at[pl.ds(wid * EPW, EPW)])


@functools.partial(
    pl.kernel,
    mesh=_sc_mesh,
    compiler_params=pltpu.CompilerParams(needs_layout_passes=False),
    out_type=jax.ShapeDtypeStruct((NC * N, D), jnp.float32),
    scratch_types=[
        pltpu.VMEM((K,), jnp.int32),
        pltpu.VMEM((K,), jnp.int32),
        pltpu.VMEM((K,), jnp.int32),
        pltpu.VMEM((K,), jnp.int32),
        pltpu.VMEM((K,), jnp.int32),
        pltpu.VMEM((K,), jnp.int32),
        pltpu.VMEM((K,), jnp.float32),
        pltpu.VMEM((K,), jnp.float32),
        pltpu.VMEM((K,), jnp.float32),
        pltpu.VMEM((K,), jnp.float32),
        pltpu.VMEM((K, D), jnp.float32),
        pltpu.VMEM((K, D), jnp.float32),
        pltpu.VMEM((K, D), jnp.float32),
        pltpu.VMEM((K, D), jnp.float32),
        pltpu.VMEM_SHARED((N, D), jnp.float32),
        pltpu.SemaphoreType.DMA,
        pltpu.SemaphoreType.DMA,
        pltpu.SemaphoreType.DMA,
        pltpu.SemaphoreType.DMA,
        pltpu.SemaphoreType.DMA,
        pltpu.SemaphoreType.DMA,
    ],
)
def _agg_sc(hw, src, dst, we, out,
            is0, is1, id0, id1, sx0, sx1, wv0, wv1, ws0, ws1,
            rin0, rin1, rout0, rout1, acc,
            isem0, isem1, gsem0, gsem1, ssem0, ssem1):
    cid = lax.axis_index("c")
    sid = lax.axis_index("s")
    wid = sid * NC + cid
    is_ = (is0, is1)
    id_ = (id0, id1)
    sx = (sx0, sx1)
    wv = (wv0, wv1)
    ws = (ws0, ws1)
    rin = (rin0, rin1)
    rout = (rout0, rout1)
    isem = (isem0, isem1)
    gsem = (gsem0, gsem1)
    ssem = (ssem0, ssem1)

    def zfill(i, c2):
        for c in range(D // 16):
            rout0[i, pl.ds(c * 16, 16)] = jnp.zeros((16,), jnp.float32)
            rout1[i, pl.ds(c * 16, 16)] = jnp.zeros((16,), jnp.float32)
        return c2

    lax.fori_loop(0, K, zfill, 0)
    r0 = sid * RPT
    for t in range(RPT // K):
        pltpu.sync_copy(rout0, acc.at[pl.ds(r0 + t * K, K)])
    pltpu.sync_copy(rout0.at[pl.ds(0, RPT - (RPT // K) * K)],
                    acc.at[pl.ds(r0 + (RPT // K) * K,
                                 RPT - (RPT // K) * K)])

    @pl.when(sid == NS - 1)
    def _():
        pltpu.sync_copy(rout0.at[pl.ds(0, 16)],
                        acc.at[pl.ds(NS * RPT, N - NS * RPT)])

    plsc.subcore_barrier()

    lanes = lax.iota(jnp.int32, 16)
    for t in range(2):
        for q in range(K // 16):
            sx[t][pl.ds(q * 16, 16)] = lanes + q * 16
        pltpu.async_copy(rout[t], acc.at[sx[t]], ssem[t], add=True)

    def issue_idx(j, t):
        base = wid * EPW + j * K
        pltpu.async_copy(src.at[pl.ds(base, K)], is_[t], isem[t])
        pltpu.async_copy(dst.at[pl.ds(base, K)], id_[t], isem[t])
        pltpu.async_copy(we.at[pl.ds(base, K)], wv[t], isem[t])

    def wait_idx(t):
        pltpu.make_async_copy(src.at[pl.ds(0, K)], is_[t], isem[t]).wait()
        pltpu.make_async_copy(dst.at[pl.ds(0, K)], id_[t], isem[t]).wait()
        pltpu.make_async_copy(we.at[pl.ds(0, K)], wv[t], isem[t]).wait()

    def issue_gather(t):
        pltpu.async_copy(hw.at[is_[t]], rin[t], gsem[t])

    def wait_gather(t):
        pltpu.make_async_copy(hw.at[is_[t]], rin[t], gsem[t]).wait()

    def wait_scatter(t):
        pltpu.make_async_copy(rout[t], acc.at[sx[t]], ssem[t]).wait()

    def step(t, nxt):
        wait_gather(t)
        wait_scatter(t)
        for q in range(K // 16):
            sx[t][pl.ds(q * 16, 16)] = id_[t][pl.ds(q * 16, 16)]
            ws[t][pl.ds(q * 16, 16)] = wv[t][pl.ds(q * 16, 16)]
        if nxt is not None:
            issue_idx(nxt, t)

        def group_body(g, c2):
            wvec = ws[t][pl.ds(g * 16, 16)]
            for b in range(4):
                wsp = [wvec[b * 4 + i] for i in range(4)]
                for c in range(D // 16):
                    sl = pl.ds(c * 16, 16)
                    for i in range(4):
                        e = g * 16 + b * 4 + i
                        rout[t][e, sl] = rin[t][e, sl] * wsp[i]
            return c2

        lax.fori_loop(0, K // 16, group_body, 0)
        pltpu.async_copy(rout[t], acc.at[sx[t]], ssem[t], add=True)

    issue_idx(0, 0)
    issue_idx(1, 1)
    wait_idx(0)
    issue_gather(0)

    def pair_body(i, c2):
        j0 = 2 * i
        wait_idx(1)
        issue_gather(1)
        step(0, j0 + 2)
        wait_idx(0)
        issue_gather(0)
        step(1, j0 + 3)
        return c2

    lax.fori_loop(0, NCHUNK // 2 - 1, pair_body, 0)
    wait_idx(1)
    issue_gather(1)
    step(0, None)
    step(1, None)
    wait_scatter(0)
    wait_scatter(1)
    plsc.subcore_barrier()

    pltpu.sync_copy(acc.at[pl.ds(sid * RPT, RPT)],
                    out.at[pl.ds(cid * N + sid * RPT, RPT)])

    @pl.when(sid == NS - 1)
    def _():
        pltpu.sync_copy(acc.at[pl.ds(NS * RPT, N - NS * RPT)],
                        out.at[pl.ds(cid * N + NS * RPT, N - NS * RPT)])


_BN = 2000


def _mm0_body(x_ref, w_ref, o_ref):
    o_ref[...] = jnp.dot(x_ref[...], w_ref[...],
                         preferred_element_type=jnp.float32)


_mm0 = pl.pallas_call(
    _mm0_body,
    grid=(N // _BN,),
    in_specs=[
        pl.BlockSpec((_BN, D), lambda i: (i, 0)),
        pl.BlockSpec((D, D), lambda i: (0, 0)),
    ],
    out_specs=pl.BlockSpec((_BN, D), lambda i: (i, 0)),
    out_shape=jax.ShapeDtypeStruct((N, D), jnp.float32),
)


def _fused_body(p_ref, b_ref, w_ref, o_ref):
    h = jnp.maximum(p_ref[0] + p_ref[1] + b_ref[...], 0.0)
    o_ref[...] = jnp.dot(h, w_ref[...], preferred_element_type=jnp.float32)


_fused = pl.pallas_call(
    _fused_body,
    grid=(N // _BN,),
    in_specs=[
        pl.BlockSpec((2, _BN, D), lambda i: (0, i, 0)),
        pl.BlockSpec((1, D), lambda i: (0, 0)),
        pl.BlockSpec((D, D), lambda i: (0, 0)),
    ],
    out_specs=pl.BlockSpec((_BN, D), lambda i: (i, 0)),
    out_shape=jax.ShapeDtypeStruct((N, D), jnp.float32),
)


def _final_body(p_ref, b_ref, o_ref):
    o_ref[...] = p_ref[0] + p_ref[1] + b_ref[...]


_final = pl.pallas_call(
    _final_body,
    grid=(N // _BN,),
    in_specs=[
        pl.BlockSpec((2, _BN, D), lambda i: (0, i, 0)),
        pl.BlockSpec((1, D), lambda i: (0, 0)),
    ],
    out_specs=pl.BlockSpec((_BN, D), lambda i: (i, 0)),
    out_shape=jax.ShapeDtypeStruct((N, D), jnp.float32),
)


def _wcalc_body(d2_ref, ea_ref, eb_ref, o_ref):
    li = pl.program_id(0)
    dist = jnp.sqrt(d2_ref[...])
    w = jax.nn.sigmoid(ea_ref[li] * dist + eb_ref[li])
    rows_i = lax.broadcasted_iota(jnp.int32, (E_PAD // D, D), 0)
    o_ref[0] = jnp.where(rows_i < E // D, w, 0.0)


_wcalc = pl.pallas_call(
    _wcalc_body,
    grid=(L3,),
    in_specs=[
        pl.BlockSpec((E_PAD // D, D), lambda l: (0, 0)),
        pl.BlockSpec(memory_space=pltpu.SMEM),
        pl.BlockSpec(memory_space=pltpu.SMEM),
    ],
    out_specs=pl.BlockSpec((1, E_PAD // D, D), lambda l: (l, 0, 0)),
    out_shape=jax.ShapeDtypeStruct((L3, E_PAD // D, D), jnp.float32),
)


def kernel(x, adj_t, embeddings, Ws, bs, ea, eb):
    pad = (lax.iota(jnp.int32, PAD) * 97) % N
    src = jnp.concatenate([adj_t[0], pad])
    dst = jnp.concatenate([adj_t[1], pad])

    d2 = _dist2_sc(embeddings, src, dst)
    w3 = _wcalc(d2.reshape(E_PAD // D, D), ea, eb)
    w3 = w3.reshape(L3, E_PAD)

    hw = _mm0(x, Ws[0])
    p = _agg_sc(hw, src, dst, w3[0]).reshape(NC, N, D)
    hw = _fused(p, bs[0].reshape(1, D), Ws[1])
    p = _agg_sc(hw, src, dst, w3[1]).reshape(NC, N, D)
    hw = _fused(p, bs[1].reshape(1, D), Ws[2])
    p = _agg_sc(hw, src, dst, w3[2]).reshape(NC, N, D)
    return _final(p, bs[2].reshape(1, D))


# speedup vs baseline: 1.0644x; 1.0297x over previous
import functools

import jax
import jax.numpy as jnp
from jax import lax
from jax.experimental import pallas as pl
from jax.experimental.pallas import tpu as pltpu
from jax.experimental.pallas import tpu_sc as plsc

N = 10000
E = 320000
D = 128
L3 = 3

NC = 2
NS = 16
NW = NC * NS
K = 80
NCHUNK = 128
EPW = K * NCHUNK
E_PAD = EPW * NW
PAD = E_PAD - E
RPT = 624
ZB = 208
ROWB = K * D * 4
KD = 64
NCHUNK_D = EPW // KD

_sc_mesh = plsc.VectorSubcoreMesh(core_axis_name="c", subcore_axis_name="s")


@functools.partial(
    pl.kernel,
    mesh=_sc_mesh,
    compiler_params=pltpu.CompilerParams(needs_layout_passes=False),
    out_type=jax.ShapeDtypeStruct((E_PAD,), jnp.float32),
    scratch_types=[
        pltpu.VMEM((KD,), jnp.int32),
        pltpu.VMEM((KD,), jnp.int32),
        pltpu.VMEM((KD,), jnp.int32),
        pltpu.VMEM((KD,), jnp.int32),
        pltpu.VMEM((KD, D), jnp.float32),
        pltpu.VMEM((KD, D), jnp.float32),
        pltpu.VMEM((KD, D), jnp.float32),
        pltpu.VMEM((KD, D), jnp.float32),
        pltpu.VMEM((EPW,), jnp.float32),
        pltpu.VMEM((16, 17), jnp.float32),
        pltpu.VMEM_SHARED((N, D), jnp.float32),
        pltpu.SemaphoreType.DMA,
        pltpu.SemaphoreType.DMA,
        pltpu.SemaphoreType.DMA,
        pltpu.SemaphoreType.DMA,
    ],
)
def _dist2_sc(emb, src, dst, d2_out,
              is0, is1, id0, id1, rs0, rs1, rd0, rd1, d2a, tb, emb_sh,
              isem0, isem1, gsem0, gsem1):
    sid = lax.axis_index("s")
    wid = sid * NC + lax.axis_index("c")

    r0 = sid * RPT
    pltpu.sync_copy(emb.at[pl.ds(r0, RPT)], emb_sh.at[pl.ds(r0, RPT)])

    @pl.when(sid == NS - 1)
    def _():
        pltpu.sync_copy(emb.at[pl.ds(NS * RPT, N - NS * RPT)],
                        emb_sh.at[pl.ds(NS * RPT, N - NS * RPT)])

    plsc.subcore_barrier()
    is_ = (is0, is1)
    id_ = (id0, id1)
    rs = (rs0, rs1)
    rd = (rd0, rd1)
    isem = (isem0, isem1)
    gsem = (gsem0, gsem1)
    lanes = lax.iota(jnp.int32, 16)

    def issue_idx(j, t):
        base = wid * EPW + j * KD
        pltpu.async_copy(src.at[pl.ds(base, KD)], is_[t], isem[t])
        pltpu.async_copy(dst.at[pl.ds(base, KD)], id_[t], isem[t])

    def wait_idx(t):
        pltpu.make_async_copy(src.at[pl.ds(0, KD)], is_[t], isem[t]).wait()
        pltpu.make_async_copy(dst.at[pl.ds(0, KD)], id_[t], isem[t]).wait()

    def issue_gather(t):
        pltpu.async_copy(emb_sh.at[is_[t]], rs[t], gsem[t])
        pltpu.async_copy(emb_sh.at[id_[t]], rd[t], gsem[t])

    def wait_gather(t):
        pltpu.make_async_copy(emb_sh.at[is_[t]], rs[t], gsem[t]).wait()
        pltpu.make_async_copy(emb_sh.at[id_[t]], rd[t], gsem[t]).wait()

    def compute(j, t):
        def group_body(g, c2):
            for b in range(4):
                accs = [jnp.zeros((16,), jnp.float32) for _ in range(4)]
                for c in range(D // 16):
                    sl = pl.ds(c * 16, 16)
                    for i in range(4):
                        e = g * 16 + b * 4 + i
                        df = rs[t][e, sl] - rd[t][e, sl]
                        accs[i] = accs[i] + df * df
                for i in range(4):
                    plsc.store_scatter(
                        tb,
                        [lanes, jnp.full((16,), b * 4 + i, jnp.int32)],
                        accs[i])
            s = tb[0, pl.ds(0, 16)]
            for r in range(1, 16):
                s = s + tb[r, pl.ds(0, 16)]
            d2a[pl.ds(j * KD + g * 16, 16)] = s
            return c2

        lax.fori_loop(0, KD // 16, group_body, 0)

    issue_idx(0, 0)
    issue_idx(1, 1)
    wait_idx(0)
    issue_gather(0)
    wait_idx(1)
    issue_gather(1)

    def pair_body(i, c2):
        j0 = 2 * i
        wait_gather(0)
        issue_idx(j0 + 2, 0)
        compute(j0, 0)
        wait_gather(1)
        issue_idx(j0 + 3, 1)
        wait_idx(0)
        issue_gather(0)
        compute(j0 + 1, 1)
        wait_idx(1)
        issue_gather(1)
        return c2

    lax.fori_loop(0, NCHUNK_D // 2 - 1, pair_body, 0)
    wait_gather(0)
    compute(NCHUNK_D - 2, 0)
    wait_gather(1)
    compute(NCHUNK_D - 1, 1)
    pltpu.sync_copy(d2a, d2_out.at[pl.ds(wid * EPW, EPW)])


@functools.partial(
    pl.kernel,
    mesh=_sc_mesh,
    compiler_params=pltpu.CompilerParams(needs_layout_passes=False),
    out_type=jax.ShapeDtypeStruct((NC * N, D), jnp.float32),
    scratch_types=[
        pltpu.VMEM((K,), jnp.int32),
        pltpu.VMEM((K,), jnp.int32),
        pltpu.VMEM((K,), jnp.int32),
        pltpu.VMEM((K,), jnp.int32),
        pltpu.VMEM((K,), jnp.int32),
        pltpu.VMEM((K,), jnp.int32),
        pltpu.VMEM((K,), jnp.float32),
        pltpu.VMEM((K,), jnp.float32),
        pltpu.VMEM((K,), jnp.float32),
        pltpu.VMEM((K,), jnp.float32),
        pltpu.VMEM((K, D), jnp.float32),
        pltpu.VMEM((K, D), jnp.float32),
        pltpu.VMEM((K, D), jnp.float32),
        pltpu.VMEM((K, D), jnp.float32),
        pltpu.VMEM_SHARED((N, D), jnp.float32),
        pltpu.SemaphoreType.DMA,
        pltpu.SemaphoreType.DMA,
        pltpu.SemaphoreType.DMA,
        pltpu.SemaphoreType.DMA,
        pltpu.SemaphoreType.DMA,
        pltpu.SemaphoreType.DMA,
    ],
)
def _agg_sc(hw, src, dst, we, out,
            is0, is1, id0, id1, sx0, sx1, wv0, wv1, ws0, ws1,
            rin0, rin1, rout0, rout1, acc,
            isem0, isem1, gsem0, gsem1, ssem0, ssem1):
    cid = lax.axis_index("c")
    sid = lax.axis_index("s")
    wid = sid * NC + cid
    is_ = (is0, is1)
    id_ = (id0, id1)
    sx = (sx0, sx1)
    wv = (wv0, wv1)
    ws = (ws0, ws1)
    rin = (rin0, rin1)
    rout = (rout0, rout1)
    isem = (isem0, isem1)
    gsem = (gsem0, gsem1)
    ssem = (ssem0, ssem1)

    def zfill(i, c2):
        for c in range(D // 16):
            rout0[i, pl.ds(c * 16, 16)] = jnp.zeros((16,), jnp.float32)
            rout1[i, pl.ds(c * 16, 16)] = jnp.zeros((16,), jnp.float32)
        return c2

    lax.fori_loop(0, K, zfill, 0)
    r0 = sid * RPT
    for t in range(RPT // K):
        pltpu.sync_copy(rout0, acc.at[pl.ds(r0 + t * K, K)])
    pltpu.sync_copy(rout0.at[pl.ds(0, RPT - (RPT // K) * K)],
                    acc.at[pl.ds(r0 + (RPT // K) * K,
                                 RPT - (RPT // K) * K)])

    @pl.when(sid == NS - 1)
    def _():
        pltpu.sync_copy(rout0.at[pl.ds(0, 16)],
                        acc.at[pl.ds(NS * RPT, N - NS * RPT)])

    plsc.subcore_barrier()

    lanes = lax.iota(jnp.int32, 16)
    for t in range(2):
        for q in range(K // 16):
            sx[t][pl.ds(q * 16, 16)] = lanes + q * 16
        pltpu.async_copy(rout[t], acc.at[sx[t]], ssem[t], add=True)

    def issue_idx(j, t):
        base = wid * EPW + j * K
        pltpu.async_copy(src.at[pl.ds(base, K)], is_[t], isem[t])
        pltpu.async_copy(dst.at[pl.ds(base, K)], id_[t], isem[t])
        pltpu.async_copy(we.at[pl.ds(base, K)], wv[t], isem[t])

    def wait_idx(t):
        pltpu.make_async_copy(src.at[pl.ds(0, K)], is_[t], isem[t]).wait()
        pltpu.make_async_copy(dst.at[pl.ds(0, K)], id_[t], isem[t]).wait()
        pltpu.make_async_copy(we.at[pl.ds(0, K)], wv[t], isem[t]).wait()

    def issue_gather(t):
        pltpu.async_copy(hw.at[is_[t]], rin[t], gsem[t])

    def wait_gather(t):
        pltpu.make_async_copy(hw.at[is_[t]], rin[t], gsem[t]).wait()

    def wait_scatter(t):
        pltpu.make_async_copy(rout[t], acc.at[sx[t]], ssem[t]).wait()

    def step(t, nxt):
        wait_gather(t)
        wait_scatter(t)
        for q in range(K // 16):
            sx[t][pl.ds(q * 16, 16)] = id_[t][pl.ds(q * 16, 16)]
            ws[t][pl.ds(q * 16, 16)] = wv[t][pl.ds(q * 16, 16)]
        if nxt is not None:
            issue_idx(nxt, t)

        def group_body(g, c2):
            wvec = ws[t][pl.ds(g * 16, 16)]
            for i in range(16):
                e = g * 16 + i
                w = wvec[i]
                for c in range(D // 16):
                    sl = pl.ds(c * 16, 16)
                    rout[t][e, sl] = rin[t][e, sl] * w
            return c2

        lax.fori_loop(0, K // 16, group_body, 0)
        pltpu.async_copy(rout[t], acc.at[sx[t]], ssem[t], add=True)

    issue_idx(0, 0)
    issue_idx(1, 1)
    wait_idx(0)
    issue_gather(0)

    def pair_body(i, c2):
        j0 = 2 * i
        wait_idx(1)
        issue_gather(1)
        step(0, j0 + 2)
        wait_idx(0)
        issue_gather(0)
        step(1, j0 + 3)
        return c2

    lax.fori_loop(0, NCHUNK // 2 - 1, pair_body, 0)
    wait_idx(1)
    issue_gather(1)
    step(0, None)
    step(1, None)
    wait_scatter(0)
    wait_scatter(1)
    plsc.subcore_barrier()

    pltpu.sync_copy(acc.at[pl.ds(sid * RPT, RPT)],
                    out.at[pl.ds(cid * N + sid * RPT, RPT)])

    @pl.when(sid == NS - 1)
    def _():
        pltpu.sync_copy(acc.at[pl.ds(NS * RPT, N - NS * RPT)],
                        out.at[pl.ds(cid * N + NS * RPT, N - NS * RPT)])


_BN = 2000


def _mm0_body(x_ref, w_ref, o_ref):
    o_ref[...] = jnp.dot(x_ref[...], w_ref[...],
                         preferred_element_type=jnp.float32)


_mm0 = pl.pallas_call(
    _mm0_body,
    grid=(N // _BN,),
    in_specs=[
        pl.BlockSpec((_BN, D), lambda i: (i, 0)),
        pl.BlockSpec((D, D), lambda i: (0, 0)),
    ],
    out_specs=pl.BlockSpec((_BN, D), lambda i: (i, 0)),
    out_shape=jax.ShapeDtypeStruct((N, D), jnp.float32),
)


def _fused_body(p_ref, b_ref, w_ref, o_ref):
    h = jnp.maximum(p_ref[0] + p_ref[1] + b_ref[...], 0.0)
    o_ref[...] = jnp.dot(h, w_ref[...], preferred_element_type=jnp.float32)


_fused = pl.pallas_call(
    _fused_body,
    grid=(N // _BN,),
    in_specs=[
        pl.BlockSpec((2, _BN, D), lambda i: (0, i, 0)),
        pl.BlockSpec((1, D), lambda i: (0, 0)),
        pl.BlockSpec((D, D), lambda i: (0, 0)),
    ],
    out_specs=pl.BlockSpec((_BN, D), lambda i: (i, 0)),
    out_shape=jax.ShapeDtypeStruct((N, D), jnp.float32),
)


def _final_body(p_ref, b_ref, o_ref):
    o_ref[...] = p_ref[0] + p_ref[1] + b_ref[...]


_final = pl.pallas_call(
    _final_body,
    grid=(N // _BN,),
    in_specs=[
        pl.BlockSpec((2, _BN, D), lambda i: (0, i, 0)),
        pl.BlockSpec((1, D), lambda i: (0, 0)),
    ],
    out_specs=pl.BlockSpec((_BN, D), lambda i: (i, 0)),
    out_shape=jax.ShapeDtypeStruct((N, D), jnp.float32),
)


def _wcalc_body(d2_ref, ea_ref, eb_ref, o_ref):
    li = pl.program_id(0)
    dist = jnp.sqrt(d2_ref[...])
    w = jax.nn.sigmoid(ea_ref[li] * dist + eb_ref[li])
    rows_i = lax.broadcasted_iota(jnp.int32, (E_PAD // D, D), 0)
    o_ref[0] = jnp.where(rows_i < E // D, w, 0.0)


_wcalc = pl.pallas_call(
    _wcalc_body,
    grid=(L3,),
    in_specs=[
        pl.BlockSpec((E_PAD // D, D), lambda l: (0, 0)),
        pl.BlockSpec(memory_space=pltpu.SMEM),
        pl.BlockSpec(memory_space=pltpu.SMEM),
    ],
    out_specs=pl.BlockSpec((1, E_PAD // D, D), lambda l: (l, 0, 0)),
    out_shape=jax.ShapeDtypeStruct((L3, E_PAD // D, D), jnp.float32),
)


def kernel(x, adj_t, embeddings, Ws, bs, ea, eb):
    pad = (lax.iota(jnp.int32, PAD) * 97) % N
    src = jnp.concatenate([adj_t[0], pad])
    dst = jnp.concatenate([adj_t[1], pad])

    d2 = _dist2_sc(embeddings, src, dst)
    w3 = _wcalc(d2.reshape(E_PAD // D, D), ea, eb)
    w3 = w3.reshape(L3, E_PAD)

    hw = _mm0(x, Ws[0])
    p = _agg_sc(hw, src, dst, w3[0]).reshape(NC, N, D)
    hw = _fused(p, bs[0].reshape(1, D), Ws[1])
    p = _agg_sc(hw, src, dst, w3[1]).reshape(NC, N, D)
    hw = _fused(p, bs[1].reshape(1, D), Ws[2])
    p = _agg_sc(hw, src, dst, w3[2]).reshape(NC, N, D)
    return _final(p, bs[2].reshape(1, D))
